# baseline (speedup 1.0000x reference)
import functools

import jax
import jax.numpy as jnp
from jax import lax
from jax.experimental import pallas as pl
from jax.experimental.pallas import tpu as pltpu
from jax.experimental.pallas import tpu_sc as plsc

N_EXP = 8
K = 2
D = 1024
F = 4096

TB = 512
BT = 256
CAP = 4096 * K + N_EXP * BT
NB = CAP // BT


def _router_body(x_ref, wr_ref, logits_ref, sel_ref, selT_ref, wT_ref,
                 xcopy_ref):
    x = x_ref[...]
    xcopy_ref[...] = x
    logits = jax.lax.dot_general(
        x, wr_ref[...], (((1,), (0,)), ((), ())),
        preferred_element_type=jnp.float32)
    logits_ref[...] = logits
    probs = jax.nn.softmax(logits, axis=1)
    i0 = jnp.argmax(probs, axis=1)
    lane = jax.lax.broadcasted_iota(jnp.int32, probs.shape, 1)
    m0 = lane == i0[:, None]
    w0 = jnp.max(probs, axis=1)
    probs_m = jnp.where(m0, -jnp.inf, probs)
    i1 = jnp.argmax(probs_m, axis=1)
    w1 = jnp.max(probs_m, axis=1)
    sel_ref[...] = jnp.stack([i0, i1], axis=1).astype(jnp.int32)
    selT_ref[...] = jnp.stack([i0, i1], axis=0).astype(jnp.int32)
    wT_ref[...] = jnp.stack([w0, w1], axis=0)


def _cumsum_body(sel_ref, cexcl_ref, counts_ref, carry):
    t = pl.program_id(0)

    @pl.when(t == 0)
    def _():
        carry[...] = jnp.zeros_like(carry)

    sel = sel_ref[...]
    lane = jax.lax.broadcasted_iota(jnp.int32, (TB, N_EXP), 1)
    onehot = ((lane == sel[:, 0][:, None]) |
              (lane == sel[:, 1][:, None])).astype(jnp.float32)
    r = jax.lax.broadcasted_iota(jnp.int32, (TB, TB), 0)
    c = jax.lax.broadcasted_iota(jnp.int32, (TB, TB), 1)
    tril_strict = (c < r).astype(jnp.float32)
    local = jax.lax.dot_general(
        tril_strict, onehot, (((1,), (0,)), ((), ())),
        preferred_element_type=jnp.float32)
    cexcl_ref[...] = local + carry[...]
    carry[...] += jnp.sum(onehot, axis=0, keepdims=True)

    @pl.when(t == pl.num_programs(0) - 1)
    def _():
        counts_ref[...] = carry[...]


def _dispatch_body(counts_ref, cexcl_ref, sel_ref, posT_ref, be_ref):
    counts = counts_ref[...]
    aligned = jnp.ceil(counts / BT) * BT
    u = jax.lax.broadcasted_iota(jnp.int32, (N_EXP, N_EXP), 0)
    v = jax.lax.broadcasted_iota(jnp.int32, (N_EXP, N_EXP), 1)
    incl_tri = (u <= v).astype(jnp.float32)
    incl = jax.lax.dot_general(
        aligned, incl_tri, (((1,), (0,)), ((), ())),
        preferred_element_type=jnp.float32)
    excl = incl - aligned
    sel = sel_ref[...]
    cexcl = cexcl_ref[...]
    lane = jax.lax.broadcasted_iota(jnp.int32, (4096, N_EXP), 1)
    base = excl + cexcl
    p0 = jnp.sum(jnp.where(lane == sel[:, 0][:, None], base, 0.0), axis=1)
    p1 = jnp.sum(jnp.where(lane == sel[:, 1][:, None], base, 0.0), axis=1)
    posT_ref[...] = jnp.stack([p0, p1], axis=0).astype(jnp.int32)
    incl_sub = jnp.transpose(incl)
    bl = jax.lax.broadcasted_iota(jnp.int32, (N_EXP, 64), 1)
    m = ((bl * BT).astype(jnp.float32) >= incl_sub).astype(jnp.float32)
    be = jax.lax.dot_general(
        jnp.ones((1, N_EXP), jnp.float32), m, (((1,), (0,)), ((), ())),
        preferred_element_type=jnp.float32)
    be_ref[...] = jnp.minimum(be, N_EXP - 1).astype(jnp.int32)


FB = 2048
NF = F // FB

NW = 32
RPW = CAP // NW
GCH = 16
TPW = 4096 // NW
CCH = 16

_SC_MESH = plsc.VectorSubcoreMesh(core_axis_name="c", subcore_axis_name="s")


def _wid():
    return lax.axis_index("s") * 2 + lax.axis_index("c")


@functools.partial(
    pl.kernel,
    out_type=[jax.ShapeDtypeStruct((CAP,), jnp.int32),
              jax.ShapeDtypeStruct((CAP,), jnp.float32)],
    scratch_types=[pltpu.VMEM((K, 4096), jnp.int32),
                   pltpu.VMEM((K, 4096), jnp.float32),
                   pltpu.VMEM((CAP,), jnp.int32),
                   pltpu.VMEM((CAP,), jnp.float32)],
    mesh=_SC_MESH,
    compiler_params=pltpu.CompilerParams(needs_layout_passes=False),
)
def _sc_invert(posT_hbm, wT_hbm, tok_hbm, wout_hbm, pos_v, w_v, tokbuf, wbuf):
    @pl.when(_wid() == 0)
    def _():
        pltpu.sync_copy(posT_hbm, pos_v)
        pltpu.sync_copy(wT_hbm, w_v)

        def zero(i, _):
            tokbuf[pl.ds(i * 16, 16)] = jnp.zeros((16,), jnp.int32)
            wbuf[pl.ds(i * 16, 16)] = jnp.zeros((16,), jnp.float32)
            return 0

        lax.fori_loop(0, CAP // 16, zero, 0)

        def scat(k):
            def body(i, _):
                idxv = pos_v[k, pl.ds(i * 16, 16)]
                tokv = lax.iota(jnp.int32, 16) + i * 16
                plsc.store_scatter(tokbuf, [idxv], tokv)
                plsc.store_scatter(wbuf, [idxv], w_v[k, pl.ds(i * 16, 16)])
                return 0
            lax.fori_loop(0, 4096 // 16, body, 0)

        scat(0)
        scat(1)
        pltpu.sync_copy(tokbuf, tok_hbm)
        pltpu.sync_copy(wbuf, wout_hbm)


@functools.partial(
    pl.kernel,
    out_type=jax.ShapeDtypeStruct((CAP, D), jnp.float32),
    scratch_types=[pltpu.VMEM((RPW,), jnp.int32),
                   [pltpu.VMEM((GCH,), jnp.int32) for _ in range(6)],
                   [pltpu.VMEM((GCH, D), jnp.float32) for _ in range(6)],
                   [pltpu.SemaphoreType.DMA for _ in range(6)],
                   [pltpu.SemaphoreType.DMA for _ in range(6)]],
    mesh=_SC_MESH,
    compiler_params=pltpu.CompilerParams(needs_layout_passes=False),
)
def _sc_gather(xs_hbm, tok_hbm, xsort_hbm, idx_v, idxcs, bufs, gsems, wsems):
    NBUF, L = 6, 5
    base = _wid() * RPW
    pltpu.sync_copy(tok_hbm.at[pl.ds(base, RPW)], idx_v)
    nch = RPW // GCH

    def gather(c):
        b = c % NBUF
        idxcs[b][...] = idx_v[pl.ds(c * GCH, GCH)]
        return pltpu.async_copy(xs_hbm.at[idxcs[b]], bufs[b], gsems[b])

    gcps, wcps = {}, {}
    for c in range(min(L, nch)):
        gcps[c] = gather(c)
    for c in range(nch):
        gcps[c].wait()
        wcps[c] = pltpu.async_copy(
            bufs[c % NBUF], xsort_hbm.at[pl.ds(base + c * GCH, GCH)],
            wsems[c % NBUF])
        n = c + L
        if n < nch:
            if n >= NBUF:
                wcps[n - NBUF].wait()
            gcps[n] = gather(n)
    for c in range(max(0, nch - NBUF), nch):
        if c in wcps:
            wcps[c].wait()


@functools.partial(
    pl.kernel,
    out_type=jax.ShapeDtypeStruct((4096, D), jnp.float32),
    scratch_types=[pltpu.VMEM((TPW,), jnp.int32),
                   pltpu.VMEM((TPW,), jnp.int32),
                   [pltpu.VMEM((CCH,), jnp.int32) for _ in range(3)],
                   [pltpu.VMEM((CCH,), jnp.int32) for _ in range(3)],
                   [pltpu.VMEM((CCH, D), jnp.float32) for _ in range(3)],
                   [pltpu.VMEM((CCH, D), jnp.float32) for _ in range(3)],
                   [pltpu.SemaphoreType.DMA for _ in range(3)],
                   [pltpu.SemaphoreType.DMA for _ in range(3)],
                   [pltpu.SemaphoreType.DMA for _ in range(3)]],
    mesh=_SC_MESH,
    compiler_params=pltpu.CompilerParams(needs_layout_passes=False),
)
def _sc_combine(pos0_hbm, pos1_hbm, osort_hbm, res_hbm,
                p0_v, p1_v, idxas, idxbs, bufas, bufbs, gasems, gbsems, wsems):
    NSET, L = 3, 2
    base = _wid() * TPW
    pltpu.sync_copy(pos0_hbm.at[pl.ds(base, TPW)], p0_v)
    pltpu.sync_copy(pos1_hbm.at[pl.ds(base, TPW)], p1_v)
    nch = TPW // CCH

    def gather(c):
        s = c % NSET
        idxas[s][...] = p0_v[pl.ds(c * CCH, CCH)]
        idxbs[s][...] = p1_v[pl.ds(c * CCH, CCH)]
        return (pltpu.async_copy(osort_hbm.at[idxas[s]], bufas[s], gasems[s]),
                pltpu.async_copy(osort_hbm.at[idxbs[s]], bufbs[s], gbsems[s]))

    gcps, wcps = {}, {}
    for c in range(min(L, nch)):
        gcps[c] = gather(c)
    UNR = 8
    for c in range(nch):
        s = c % NSET
        gcps[c][0].wait()
        gcps[c][1].wait()
        bufa, bufb = bufas[s], bufbs[s]

        def add(i, _):
            for u in range(UNR):
                j = i * UNR + u
                r = j // (D // 16)
                col = (j % (D // 16)) * 16
                bufa[r, pl.ds(col, 16)] = (bufa[r, pl.ds(col, 16)] +
                                           bufb[r, pl.ds(col, 16)])
            return 0

        lax.fori_loop(0, CCH * D // 16 // UNR, add, 0)
        wcps[c] = pltpu.async_copy(
            bufa, res_hbm.at[pl.ds(base + c * CCH, CCH)], wsems[s])
        n = c + L
        if n < nch:
            if n >= NSET:
                wcps[n - NSET].wait()
            gcps[n] = gather(n)
    for c in range(max(0, nch - NSET), nch):
        if c in wcps:
            wcps[c].wait()


def _gmm_body(be_ref, x_ref, w1_ref, w2_ref, ws_ref, acc_ref, out_ref):
    f = pl.program_id(0)
    h = jax.lax.dot_general(
        x_ref[...], w1_ref[0], (((1,), (0,)), ((), ())),
        preferred_element_type=jnp.float32)
    h = jax.nn.gelu(h)
    out = jax.lax.dot_general(
        h, w2_ref[0], (((1,), (0,)), ((), ())),
        preferred_element_type=jnp.float32)
    contrib = out * ws_ref[...]

    @pl.when(f == 0)
    def _():
        out_ref[...] = contrib

    @pl.when(f != 0)
    def _():
        out_ref[...] = acc_ref[...] + contrib


def kernel(inputs, W_router, W1, W2):
    xs = inputs.reshape(-1, D)
    T = xs.shape[0]
    n_tb = T // TB

    logits, sel, selT, wT, xs_copy = pl.pallas_call(
        _router_body,
        grid=(n_tb,),
        in_specs=[
            pl.BlockSpec((TB, D), lambda t: (t, 0)),
            pl.BlockSpec((D, N_EXP), lambda t: (0, 0)),
        ],
        out_specs=[
            pl.BlockSpec((TB, N_EXP), lambda t: (t, 0)),
            pl.BlockSpec((TB, K), lambda t: (t, 0)),
            pl.BlockSpec((K, TB), lambda t: (0, t)),
            pl.BlockSpec((K, TB), lambda t: (0, t)),
            pl.BlockSpec((TB, D), lambda t: (t, 0)),
        ],
        out_shape=[
            jax.ShapeDtypeStruct((T, N_EXP), jnp.float32),
            jax.ShapeDtypeStruct((T, K), jnp.int32),
            jax.ShapeDtypeStruct((K, T), jnp.int32),
            jax.ShapeDtypeStruct((K, T), jnp.float32),
            jax.ShapeDtypeStruct((T, D), jnp.float32),
        ],
    )(xs, W_router)

    cexcl, counts = pl.pallas_call(
        _cumsum_body,
        grid=(n_tb,),
        in_specs=[pl.BlockSpec((TB, K), lambda t: (t, 0))],
        out_specs=[
            pl.BlockSpec((TB, N_EXP), lambda t: (t, 0)),
            pl.BlockSpec((1, N_EXP), lambda t: (0, 0)),
        ],
        out_shape=[
            jax.ShapeDtypeStruct((T, N_EXP), jnp.float32),
            jax.ShapeDtypeStruct((1, N_EXP), jnp.float32),
        ],
        scratch_shapes=[pltpu.VMEM((1, N_EXP), jnp.float32)],
        compiler_params=pltpu.CompilerParams(
            dimension_semantics=("arbitrary",)),
    )(sel)

    posT, be = pl.pallas_call(
        _dispatch_body,
        in_specs=[
            pl.BlockSpec((1, N_EXP), lambda: (0, 0)),
            pl.BlockSpec((T, N_EXP), lambda: (0, 0)),
            pl.BlockSpec((T, K), lambda: (0, 0)),
        ],
        out_specs=[
            pl.BlockSpec((K, T), lambda: (0, 0)),
            pl.BlockSpec((1, 64), lambda: (0, 0)),
        ],
        out_shape=[
            jax.ShapeDtypeStruct((K, T), jnp.int32),
            jax.ShapeDtypeStruct((1, 64), jnp.int32),
        ],
    )(counts, cexcl, sel)

    sorted_tok, sorted_w = _sc_invert(posT, wT)
    x_sorted = _sc_gather(xs_copy, sorted_tok)

    block_expert = be[0, :NB]

    grid_spec = pltpu.PrefetchScalarGridSpec(
        num_scalar_prefetch=1,
        grid=(NF, NB),
        in_specs=[
            pl.BlockSpec((BT, D), lambda f, b, be_r: (b, 0)),
            pl.BlockSpec((1, D, FB), lambda f, b, be_r: (be_r[b], 0, f)),
            pl.BlockSpec((1, FB, D), lambda f, b, be_r: (be_r[b], f, 0)),
            pl.BlockSpec((BT, 1), lambda f, b, be_r: (b, 0)),
            pl.BlockSpec((BT, D), lambda f, b, be_r: (b, 0)),
        ],
        out_specs=pl.BlockSpec((BT, D), lambda f, b, be_r: (b, 0)),
    )
    acc_init = jnp.zeros((CAP, D), jnp.float32)
    out_sorted = pl.pallas_call(
        _gmm_body,
        grid_spec=grid_spec,
        out_shape=jax.ShapeDtypeStruct((CAP, D), jnp.float32),
        input_output_aliases={5: 0},
        compiler_params=pltpu.CompilerParams(
            dimension_semantics=("arbitrary", "arbitrary"),
            vmem_limit_bytes=60 * 1024 * 1024,
        ),
    )(block_expert, x_sorted, W1, W2, sorted_w.reshape(CAP, 1), acc_init)

    results = _sc_combine(posT[0], posT[1], out_sorted)

    return (results.reshape(inputs.shape), logits, sel)


# speedup vs baseline: 1.3510x; 1.3510x over previous
import functools

import jax
import jax.numpy as jnp
from jax import lax
from jax.experimental import pallas as pl
from jax.experimental.pallas import tpu as pltpu
from jax.experimental.pallas import tpu_sc as plsc

N_EXP = 8
K = 2
D = 1024
F = 4096

TB = 512
BT = 256
CAP = 4096 * K + N_EXP * BT
NB = CAP // BT


def _router_body(x_ref, wr_ref, logits_ref, sel_ref, selT_ref, wT_ref,
                 xcopy_ref):
    x = x_ref[...]
    xcopy_ref[...] = x
    logits = jax.lax.dot_general(
        x, wr_ref[...], (((1,), (0,)), ((), ())),
        preferred_element_type=jnp.float32)
    logits_ref[...] = logits
    probs = jax.nn.softmax(logits, axis=1)
    i0 = jnp.argmax(probs, axis=1)
    lane = jax.lax.broadcasted_iota(jnp.int32, probs.shape, 1)
    m0 = lane == i0[:, None]
    w0 = jnp.max(probs, axis=1)
    probs_m = jnp.where(m0, -jnp.inf, probs)
    i1 = jnp.argmax(probs_m, axis=1)
    w1 = jnp.max(probs_m, axis=1)
    sel_ref[...] = jnp.stack([i0, i1], axis=1).astype(jnp.int32)
    selT_ref[...] = jnp.stack([i0, i1], axis=0).astype(jnp.int32)
    wT_ref[...] = jnp.stack([w0, w1], axis=0)


def _cumsum_body(sel_ref, cexcl_ref, counts_ref, carry):
    t = pl.program_id(0)

    @pl.when(t == 0)
    def _():
        carry[...] = jnp.zeros_like(carry)

    sel = sel_ref[...]
    lane = jax.lax.broadcasted_iota(jnp.int32, (TB, N_EXP), 1)
    onehot = ((lane == sel[:, 0][:, None]) |
              (lane == sel[:, 1][:, None])).astype(jnp.float32)
    r = jax.lax.broadcasted_iota(jnp.int32, (TB, TB), 0)
    c = jax.lax.broadcasted_iota(jnp.int32, (TB, TB), 1)
    tril_strict = (c < r).astype(jnp.float32)
    local = jax.lax.dot_general(
        tril_strict, onehot, (((1,), (0,)), ((), ())),
        preferred_element_type=jnp.float32)
    cexcl_ref[...] = local + carry[...]
    carry[...] += jnp.sum(onehot, axis=0, keepdims=True)

    @pl.when(t == pl.num_programs(0) - 1)
    def _():
        counts_ref[...] = carry[...]


def _dispatch_body(counts_ref, cexcl_ref, sel_ref, posT_ref, be_ref):
    counts = counts_ref[...]
    aligned = jnp.ceil(counts / BT) * BT
    u = jax.lax.broadcasted_iota(jnp.int32, (N_EXP, N_EXP), 0)
    v = jax.lax.broadcasted_iota(jnp.int32, (N_EXP, N_EXP), 1)
    incl_tri = (u <= v).astype(jnp.float32)
    incl = jax.lax.dot_general(
        aligned, incl_tri, (((1,), (0,)), ((), ())),
        preferred_element_type=jnp.float32)
    excl = incl - aligned
    sel = sel_ref[...]
    cexcl = cexcl_ref[...]
    lane = jax.lax.broadcasted_iota(jnp.int32, (4096, N_EXP), 1)
    base = excl + cexcl
    p0 = jnp.sum(jnp.where(lane == sel[:, 0][:, None], base, 0.0), axis=1)
    p1 = jnp.sum(jnp.where(lane == sel[:, 1][:, None], base, 0.0), axis=1)
    posT_ref[...] = jnp.stack([p0, p1], axis=0).astype(jnp.int32)
    incl_sub = jnp.transpose(incl)
    bl = jax.lax.broadcasted_iota(jnp.int32, (N_EXP, 64), 1)
    m = ((bl * BT).astype(jnp.float32) >= incl_sub).astype(jnp.float32)
    be = jax.lax.dot_general(
        jnp.ones((1, N_EXP), jnp.float32), m, (((1,), (0,)), ((), ())),
        preferred_element_type=jnp.float32)
    be_ref[...] = jnp.minimum(be, N_EXP - 1).astype(jnp.int32)


FB = 2048
NF = F // FB

NW = 32
RPW = CAP // NW
GCH = 16
TPW = 4096 // NW
CCH = 16

_SC_MESH = plsc.VectorSubcoreMesh(core_axis_name="c", subcore_axis_name="s")


def _wid():
    return lax.axis_index("s") * 2 + lax.axis_index("c")


@functools.partial(
    pl.kernel,
    out_type=jax.ShapeDtypeStruct((CAP, D), jnp.float32),
    scratch_types=[pltpu.VMEM((TPW,), jnp.int32),
                   pltpu.VMEM((TPW,), jnp.int32),
                   [pltpu.VMEM((CCH,), jnp.int32) for _ in range(3)],
                   [pltpu.VMEM((CCH,), jnp.int32) for _ in range(3)],
                   [pltpu.VMEM((CCH, D), jnp.float32) for _ in range(3)],
                   [pltpu.SemaphoreType.DMA for _ in range(3)],
                   [pltpu.SemaphoreType.DMA for _ in range(3)],
                   [pltpu.SemaphoreType.DMA for _ in range(3)]],
    mesh=_SC_MESH,
    compiler_params=pltpu.CompilerParams(needs_layout_passes=False),
)
def _sc_scatter(xs_hbm, pos0_hbm, pos1_hbm, xsort_hbm,
                p0_v, p1_v, idxas, idxbs, bufs, lsems, asems, bsems):
    NSET, L = 3, 2
    base = _wid() * TPW
    pltpu.sync_copy(pos0_hbm.at[pl.ds(base, TPW)], p0_v)
    pltpu.sync_copy(pos1_hbm.at[pl.ds(base, TPW)], p1_v)
    nch = TPW // CCH

    def load(c):
        s = c % NSET
        return pltpu.async_copy(
            xs_hbm.at[pl.ds(base + c * CCH, CCH)], bufs[s], lsems[s])

    lcps = {c: load(c) for c in range(min(L, nch))}
    acps, bcps = {}, {}
    for c in range(nch):
        s = c % NSET
        lcps[c].wait()
        idxas[s][...] = p0_v[pl.ds(c * CCH, CCH)]
        idxbs[s][...] = p1_v[pl.ds(c * CCH, CCH)]
        acps[c] = pltpu.async_copy(bufs[s], xsort_hbm.at[idxas[s]], asems[s])
        bcps[c] = pltpu.async_copy(bufs[s], xsort_hbm.at[idxbs[s]], bsems[s])
        n = c + L
        if n < nch:
            if n >= NSET:
                acps[n - NSET].wait()
                bcps[n - NSET].wait()
            lcps[n] = load(n)
    for c in range(max(0, nch - NSET), nch):
        if c in acps:
            acps[c].wait()
            bcps[c].wait()


@functools.partial(
    pl.kernel,
    out_type=jax.ShapeDtypeStruct((4096, D), jnp.float32),
    scratch_types=[pltpu.VMEM((TPW,), jnp.int32),
                   pltpu.VMEM((TPW,), jnp.int32),
                   pltpu.VMEM((TPW,), jnp.float32),
                   pltpu.VMEM((TPW,), jnp.float32),
                   [pltpu.VMEM((CCH,), jnp.int32) for _ in range(3)],
                   [pltpu.VMEM((CCH,), jnp.int32) for _ in range(3)],
                   [pltpu.VMEM((CCH, D), jnp.float32) for _ in range(3)],
                   [pltpu.VMEM((CCH, D), jnp.float32) for _ in range(3)],
                   [pltpu.SemaphoreType.DMA for _ in range(3)],
                   [pltpu.SemaphoreType.DMA for _ in range(3)],
                   [pltpu.SemaphoreType.DMA for _ in range(3)]],
    mesh=_SC_MESH,
    compiler_params=pltpu.CompilerParams(needs_layout_passes=False),
)
def _sc_combine(pos0_hbm, pos1_hbm, w0_hbm, w1_hbm, osort_hbm, res_hbm,
                p0_v, p1_v, w0_v, w1_v, idxas, idxbs, bufas, bufbs,
                gasems, gbsems, wsems):
    NSET, L = 3, 2
    base = _wid() * TPW
    pltpu.sync_copy(pos0_hbm.at[pl.ds(base, TPW)], p0_v)
    pltpu.sync_copy(pos1_hbm.at[pl.ds(base, TPW)], p1_v)
    pltpu.sync_copy(w0_hbm.at[pl.ds(base, TPW)], w0_v)
    pltpu.sync_copy(w1_hbm.at[pl.ds(base, TPW)], w1_v)
    nch = TPW // CCH

    def gather(c):
        s = c % NSET
        idxas[s][...] = p0_v[pl.ds(c * CCH, CCH)]
        idxbs[s][...] = p1_v[pl.ds(c * CCH, CCH)]
        return (pltpu.async_copy(osort_hbm.at[idxas[s]], bufas[s], gasems[s]),
                pltpu.async_copy(osort_hbm.at[idxbs[s]], bufbs[s], gbsems[s]))

    gcps, wcps = {}, {}
    for c in range(min(L, nch)):
        gcps[c] = gather(c)
    UNR = 8
    SLICES_PER_ROW = D // 16
    for c in range(nch):
        s = c % NSET
        gcps[c][0].wait()
        gcps[c][1].wait()
        bufa, bufb = bufas[s], bufbs[s]

        def fma(i, _):
            r = (i * UNR) // SLICES_PER_ROW
            rsplat = jnp.full((16,), c * CCH + r, jnp.int32)
            w0r = plsc.load_gather(w0_v, [rsplat])
            w1r = plsc.load_gather(w1_v, [rsplat])
            for u in range(UNR):
                col = ((i * UNR + u) % SLICES_PER_ROW) * 16
                bufa[r, pl.ds(col, 16)] = (
                    w0r * bufa[r, pl.ds(col, 16)] +
                    w1r * bufb[r, pl.ds(col, 16)])
            return 0

        lax.fori_loop(0, CCH * D // 16 // UNR, fma, 0)
        wcps[c] = pltpu.async_copy(
            bufa, res_hbm.at[pl.ds(base + c * CCH, CCH)], wsems[s])
        n = c + L
        if n < nch:
            if n >= NSET:
                wcps[n - NSET].wait()
            gcps[n] = gather(n)
    for c in range(max(0, nch - NSET), nch):
        if c in wcps:
            wcps[c].wait()


def _gmm_body(be_ref, x_ref, w1_ref, w2_ref, acc_ref, out_ref):
    f = pl.program_id(0)
    h = jax.lax.dot_general(
        x_ref[...], w1_ref[0], (((1,), (0,)), ((), ())),
        preferred_element_type=jnp.float32)
    h = jax.nn.gelu(h)
    contrib = jax.lax.dot_general(
        h, w2_ref[0], (((1,), (0,)), ((), ())),
        preferred_element_type=jnp.float32)

    @pl.when(f == 0)
    def _():
        out_ref[...] = contrib

    @pl.when(f != 0)
    def _():
        out_ref[...] = acc_ref[...] + contrib


def kernel(inputs, W_router, W1, W2):
    xs = inputs.reshape(-1, D)
    T = xs.shape[0]
    n_tb = T // TB

    logits, sel, selT, wT, xs_copy = pl.pallas_call(
        _router_body,
        grid=(n_tb,),
        in_specs=[
            pl.BlockSpec((TB, D), lambda t: (t, 0)),
            pl.BlockSpec((D, N_EXP), lambda t: (0, 0)),
        ],
        out_specs=[
            pl.BlockSpec((TB, N_EXP), lambda t: (t, 0)),
            pl.BlockSpec((TB, K), lambda t: (t, 0)),
            pl.BlockSpec((K, TB), lambda t: (0, t)),
            pl.BlockSpec((K, TB), lambda t: (0, t)),
            pl.BlockSpec((TB, D), lambda t: (t, 0)),
        ],
        out_shape=[
            jax.ShapeDtypeStruct((T, N_EXP), jnp.float32),
            jax.ShapeDtypeStruct((T, K), jnp.int32),
            jax.ShapeDtypeStruct((K, T), jnp.int32),
            jax.ShapeDtypeStruct((K, T), jnp.float32),
            jax.ShapeDtypeStruct((T, D), jnp.float32),
        ],
    )(xs, W_router)

    cexcl, counts = pl.pallas_call(
        _cumsum_body,
        grid=(n_tb,),
        in_specs=[pl.BlockSpec((TB, K), lambda t: (t, 0))],
        out_specs=[
            pl.BlockSpec((TB, N_EXP), lambda t: (t, 0)),
            pl.BlockSpec((1, N_EXP), lambda t: (0, 0)),
        ],
        out_shape=[
            jax.ShapeDtypeStruct((T, N_EXP), jnp.float32),
            jax.ShapeDtypeStruct((1, N_EXP), jnp.float32),
        ],
        scratch_shapes=[pltpu.VMEM((1, N_EXP), jnp.float32)],
        compiler_params=pltpu.CompilerParams(
            dimension_semantics=("arbitrary",)),
    )(sel)

    posT, be = pl.pallas_call(
        _dispatch_body,
        in_specs=[
            pl.BlockSpec((1, N_EXP), lambda: (0, 0)),
            pl.BlockSpec((T, N_EXP), lambda: (0, 0)),
            pl.BlockSpec((T, K), lambda: (0, 0)),
        ],
        out_specs=[
            pl.BlockSpec((K, T), lambda: (0, 0)),
            pl.BlockSpec((1, 64), lambda: (0, 0)),
        ],
        out_shape=[
            jax.ShapeDtypeStruct((K, T), jnp.int32),
            jax.ShapeDtypeStruct((1, 64), jnp.int32),
        ],
    )(counts, cexcl, sel)

    x_sorted = _sc_scatter(xs_copy, posT[0], posT[1])

    block_expert = be[0, :NB]

    grid_spec = pltpu.PrefetchScalarGridSpec(
        num_scalar_prefetch=1,
        grid=(NF, NB),
        in_specs=[
            pl.BlockSpec((BT, D), lambda f, b, be_r: (b, 0)),
            pl.BlockSpec((1, D, FB), lambda f, b, be_r: (be_r[b], 0, f)),
            pl.BlockSpec((1, FB, D), lambda f, b, be_r: (be_r[b], f, 0)),
            pl.BlockSpec((BT, D), lambda f, b, be_r: (b, 0)),
        ],
        out_specs=pl.BlockSpec((BT, D), lambda f, b, be_r: (b, 0)),
    )
    acc_init = jnp.zeros((CAP, D), jnp.float32)
    out_sorted = pl.pallas_call(
        _gmm_body,
        grid_spec=grid_spec,
        out_shape=jax.ShapeDtypeStruct((CAP, D), jnp.float32),
        input_output_aliases={4: 0},
        compiler_params=pltpu.CompilerParams(
            dimension_semantics=("arbitrary", "arbitrary"),
            vmem_limit_bytes=60 * 1024 * 1024,
        ),
    )(block_expert, x_sorted, W1, W2, acc_init)

    results = _sc_combine(posT[0], posT[1], wT[0], wT[1], out_sorted)

    return (results.reshape(inputs.shape), logits, sel)
